# baseline (speedup 1.0000x reference)
import functools

import jax
import jax.numpy as jnp
from jax import lax
from jax.experimental import pallas as pl
from jax.experimental.pallas import tpu as pltpu
from jax.experimental.pallas import tpu_sc as plsc

B, S, N, M, W = 16, 128, 4, 2048, 1
MC = 128
NCHUNK = M // MC // 2
L = 16

_mesh = plsc.VectorSubcoreMesh(core_axis_name="c", subcore_axis_name="s")


@functools.partial(
    pl.kernel,
    mesh=_mesh,
    out_type=jax.ShapeDtypeStruct((B, 3 * M, S), jnp.float32),
    scratch_types=[
        pltpu.VMEM((S, N), jnp.int32),
        pltpu.VMEM((3, MC, S), jnp.float32),
        pltpu.VMEM((3, MC, S), jnp.float32),
        pltpu.SemaphoreType.DMA,
        pltpu.SemaphoreType.DMA,
    ],
    compiler_params=pltpu.CompilerParams(
        needs_layout_passes=False, skip_device_barrier=True
    ),
)
def _sc_kernel(h_hbm, out_hbm, hv, buf0, buf1, sem0, sem1):
    c = lax.axis_index("c")
    b = lax.axis_index("s")
    bufs = (buf0, buf1)
    sems = (sem0, sem1)

    lanes = lax.iota(jnp.int32, L)
    vzero = jnp.zeros((L,), jnp.float32)
    k0 = jnp.zeros((L,), jnp.int32)

    hcp = pltpu.make_async_copy(h_hbm.at[b], hv, sem0)
    hcp.start()

    def zero_body(m, _):
        for buf in bufs:
            for k in range(3):
                for j in range(S // L):
                    buf[k, m, pl.ds(j * L, L)] = vzero
        return 0

    lax.fori_loop(0, MC, zero_body, 0)
    hcp.wait()

    def scatter_phase(buf, m0, val, enable):
        vval = jnp.full((L,), 1.0, jnp.float32) * val

        def n_body(n, _):
            nsplat = k0 + n

            def j_body(j, _):
                svec = lanes + j * L
                h16 = plsc.load_gather(hv, [svec, nsplat])
                rel = (h16 & (M - 1)) - m0
                ok = (rel >= 0) & (rel < MC) & enable
                plsc.addupdate_scatter(buf, [k0, rel, svec + 1], vval,
                                       mask=ok & (svec < S - 1))
                plsc.addupdate_scatter(buf, [k0 + 1, rel, svec], vval,
                                       mask=ok)
                plsc.addupdate_scatter(buf, [k0 + 2, rel, svec - 1], vval,
                                       mask=ok & (svec > 0))
                return 0

            lax.fori_loop(0, S // L, j_body, 0)
            return 0

        lax.fori_loop(0, N, n_body, 0)

    def m0_of(i):
        return (c * NCHUNK + i) * MC

    def dma(buf, sem, m0, fire):
        for k in range(3):
            cp = pltpu.make_async_copy(
                buf.at[k],
                out_hbm.at[b, pl.ds(k * M + m0, MC)],
                sem,
            )
            if fire:
                cp.start()
            else:
                cp.wait()

    def chunk_body(i, _):
        m0 = m0_of(i)
        for p in range(2):
            @pl.when(lax.rem(i, 2) == p)
            def _():
                buf, sem = bufs[p], sems[p]

                @pl.when(i >= 2)
                def _():
                    dma(buf, sem, m0 - 2 * MC, fire=False)

                def pass_body(t, _):
                    prev = t == 0
                    mm = jnp.where(prev, m0 - 2 * MC, m0)
                    vv = jnp.where(prev, -1.0, 1.0)
                    scatter_phase(buf, mm, vv, jnp.logical_or(t == 1, i >= 2))
                    return 0

                lax.fori_loop(0, 2, pass_body, 0)
                dma(buf, sem, m0, fire=True)
        return 0

    lax.fori_loop(0, NCHUNK, chunk_body, 0)

    for p in range(2):
        dma(bufs[p], sems[p], m0_of(NCHUNK - 2 + p), fire=False)


def kernel(sentencesMinHashes):
    return _sc_kernel(sentencesMinHashes)


# speedup vs baseline: 1.0263x; 1.0263x over previous
import functools

import jax
import jax.numpy as jnp
from jax import lax
from jax.experimental import pallas as pl
from jax.experimental.pallas import tpu as pltpu
from jax.experimental.pallas import tpu_sc as plsc

B, S, N, M, W = 16, 128, 4, 2048, 1
MC = 128
NCHUNK = M // MC // 2
L = 16

_mesh = plsc.VectorSubcoreMesh(core_axis_name="c", subcore_axis_name="s")


@functools.partial(
    pl.kernel,
    mesh=_mesh,
    out_type=jax.ShapeDtypeStruct((B, 3 * M, S), jnp.float32),
    scratch_types=[
        pltpu.VMEM((S * N,), jnp.int32),
        pltpu.VMEM((3, MC, S), jnp.float32),
        pltpu.VMEM((3, MC, S), jnp.float32),
        pltpu.SemaphoreType.DMA,
        pltpu.SemaphoreType.DMA,
    ],
    compiler_params=pltpu.CompilerParams(
        needs_layout_passes=False, skip_device_barrier=True
    ),
)
def _sc_kernel(h_hbm, out_hbm, hv, buf0, buf1, sem0, sem1):
    c = lax.axis_index("c")
    b = lax.axis_index("s")
    bufs = (buf0, buf1)
    sems = (sem0, sem1)

    lanes = lax.iota(jnp.int32, L)
    vzero = jnp.zeros((L,), jnp.float32)
    k0 = jnp.zeros((L,), jnp.int32)

    hcp = pltpu.make_async_copy(h_hbm.at[pl.ds(b * S * N, S * N)], hv, sem0)
    hcp.start()

    def zero_body(m, _):
        for buf in bufs:
            for k in range(3):
                for j in range(S // L):
                    buf[k, m, pl.ds(j * L, L)] = vzero
        return 0

    lax.fori_loop(0, MC, zero_body, 0)
    hcp.wait()

    def scatter_phase(buf, m0, val, enable):
        vval = jnp.full((L,), 1.0, jnp.float32) * val

        def n_body(n, _):
            nsplat = k0 + n

            def j_body(j, _):
                svec = lanes + j * L
                h16 = plsc.load_gather(hv, [svec * N + nsplat])
                rel = (h16 & (M - 1)) - m0
                ok = (rel >= 0) & (rel < MC) & enable
                plsc.addupdate_scatter(buf, [k0, rel, svec + 1], vval,
                                       mask=ok & (svec < S - 1))
                plsc.addupdate_scatter(buf, [k0 + 1, rel, svec], vval,
                                       mask=ok)
                plsc.addupdate_scatter(buf, [k0 + 2, rel, svec - 1], vval,
                                       mask=ok & (svec > 0))
                return 0

            lax.fori_loop(0, S // L, j_body, 0)
            return 0

        lax.fori_loop(0, N, n_body, 0)

    def m0_of(i):
        return (c * NCHUNK + i) * MC

    def dma(buf, sem, m0, fire):
        for k in range(3):
            cp = pltpu.make_async_copy(
                buf.at[k],
                out_hbm.at[b, pl.ds(k * M + m0, MC)],
                sem,
            )
            if fire:
                cp.start()
            else:
                cp.wait()

    def chunk_body(i, _):
        m0 = m0_of(i)
        for p in range(2):
            @pl.when(lax.rem(i, 2) == p)
            def _():
                buf, sem = bufs[p], sems[p]

                @pl.when(i >= 2)
                def _():
                    dma(buf, sem, m0 - 2 * MC, fire=False)

                def pass_body(t, _):
                    prev = t == 0
                    mm = jnp.where(prev, m0 - 2 * MC, m0)
                    vv = jnp.where(prev, -1.0, 1.0)
                    scatter_phase(buf, mm, vv, jnp.logical_or(t == 1, i >= 2))
                    return 0

                lax.fori_loop(0, 2, pass_body, 0)
                dma(buf, sem, m0, fire=True)
        return 0

    lax.fori_loop(0, NCHUNK, chunk_body, 0)

    for p in range(2):
        dma(bufs[p], sems[p], m0_of(NCHUNK - 2 + p), fire=False)


def kernel(sentencesMinHashes):
    return _sc_kernel(sentencesMinHashes.reshape(-1))


# speedup vs baseline: 1.0610x; 1.0338x over previous
import functools

import jax
import jax.numpy as jnp
from jax import lax
from jax.experimental import pallas as pl
from jax.experimental.pallas import tpu as pltpu
from jax.experimental.pallas import tpu_sc as plsc

B, S, N, M, W = 16, 128, 4, 2048, 1
MC = 128
NCHUNK = M // MC // 2
L = 16

_mesh = plsc.VectorSubcoreMesh(core_axis_name="c", subcore_axis_name="s")


@functools.partial(
    pl.kernel,
    mesh=_mesh,
    out_type=jax.ShapeDtypeStruct((B, 3 * M, S), jnp.float32),
    scratch_types=[
        pltpu.VMEM((S * N,), jnp.int32),
        pltpu.VMEM((3, MC, S), jnp.float32),
        pltpu.VMEM((3, MC, S), jnp.float32),
        pltpu.SemaphoreType.DMA,
        pltpu.SemaphoreType.DMA,
    ],
    compiler_params=pltpu.CompilerParams(
        needs_layout_passes=False, skip_device_barrier=True
    ),
)
def _sc_kernel(h_hbm, out_hbm, hv, buf0, buf1, sem0, sem1):
    c = lax.axis_index("c")
    b = lax.axis_index("s")
    bufs = (buf0, buf1)
    sems = (sem0, sem1)

    lanes = lax.iota(jnp.int32, L)
    vzero = jnp.zeros((L,), jnp.float32)
    k0 = jnp.zeros((L,), jnp.int32)

    hcp = pltpu.make_async_copy(h_hbm.at[pl.ds(b * S * N, S * N)], hv, sem0)
    hcp.start()

    def zero_buf(buf):
        def zero_body(m, _):
            for k in range(3):
                for j in range(S // L):
                    buf[k, m, pl.ds(j * L, L)] = vzero
            return 0

        lax.fori_loop(0, MC, zero_body, 0)

    zero_buf(buf0)
    hcp.wait()

    def scatter_phase(buf, m0, val, enable):
        vval = jnp.full((L,), 1.0, jnp.float32) * val

        def n_body(n, _):
            nsplat = k0 + n

            def j_body(j, _):
                svec = lanes + j * L
                h16 = plsc.load_gather(hv, [svec * N + nsplat])
                rel = (h16 & (M - 1)) - m0
                ok = (rel >= 0) & (rel < MC) & enable
                plsc.addupdate_scatter(buf, [k0, rel, svec + 1], vval,
                                       mask=ok & (svec < S - 1))
                plsc.addupdate_scatter(buf, [k0 + 1, rel, svec], vval,
                                       mask=ok)
                plsc.addupdate_scatter(buf, [k0 + 2, rel, svec - 1], vval,
                                       mask=ok & (svec > 0))
                return 0

            lax.fori_loop(0, S // L, j_body, 0)
            return 0

        lax.fori_loop(0, N, n_body, 0)

    def m0_of(i):
        return (c * NCHUNK + i) * MC

    def dma(buf, sem, m0, fire):
        for k in range(3):
            cp = pltpu.make_async_copy(
                buf.at[k],
                out_hbm.at[b, pl.ds(k * M + m0, MC)],
                sem,
            )
            if fire:
                cp.start()
            else:
                cp.wait()

    def chunk_body(i, _):
        m0 = m0_of(i)
        for p in range(2):
            @pl.when(lax.rem(i, 2) == p)
            def _():
                buf, sem = bufs[p], sems[p]

                if p == 1:
                    @pl.when(i == 1)
                    def _():
                        zero_buf(buf1)

                @pl.when(i >= 2)
                def _():
                    dma(buf, sem, m0 - 2 * MC, fire=False)

                def pass_body(t, _):
                    prev = t == 0
                    mm = jnp.where(prev, m0 - 2 * MC, m0)
                    vv = jnp.where(prev, -1.0, 1.0)
                    scatter_phase(buf, mm, vv, jnp.logical_or(t == 1, i >= 2))
                    return 0

                lax.fori_loop(0, 2, pass_body, 0)
                dma(buf, sem, m0, fire=True)
        return 0

    lax.fori_loop(0, NCHUNK, chunk_body, 0)

    for p in range(2):
        dma(bufs[p], sems[p], m0_of(NCHUNK - 2 + p), fire=False)


def kernel(sentencesMinHashes):
    return _sc_kernel(sentencesMinHashes.reshape(-1))
